# baseline (speedup 1.0000x reference)
import functools

import jax
import jax.numpy as jnp
from jax import lax
from jax.experimental import pallas as pl
from jax.experimental.pallas import tpu as pltpu
from jax.experimental.pallas import tpu_sc as plsc

B, T = 4096, 200
V, E, H = 1000000, 64, 32

NC, NS = 2, 16
NW = NC * NS
ROWS = B * T
ROWS_PER_W = ROWS // NW
CHUNK = 128
CHUNKS = ROWS_PER_W // CHUNK
NBUF = 4

BT_TILE = 4096
NB = B // BT_TILE


def _gather_body(table_hbm, idx_hbm, out_hbm, idx_v, rows_v, *sems):
    wid = lax.axis_index("s") * NC + lax.axis_index("c")
    row_base = wid * ROWS_PER_W

    pltpu.sync_copy(idx_hbm.at[wid], idx_v)

    def gather_copy(c, b):
        return pltpu.make_async_copy(
            table_hbm.at[idx_v.at[c]], rows_v.at[b], sems[b])

    for b in range(NBUF):
        gather_copy(b, b).start()

    def body(i, carry):
        for b in range(NBUF):
            c = i * NBUF + b
            gather_copy(c, b).wait()
            pltpu.sync_copy(
                rows_v.at[b],
                out_hbm.at[pl.ds(row_base + c * CHUNK, CHUNK)])
            nxt = c + NBUF

            @pl.when(nxt < CHUNKS)
            def _():
                gather_copy(nxt, b).start()
        return carry

    lax.fori_loop(0, CHUNKS // NBUF, body, 0)


@functools.cache
def _make_sc_gather():
    return pl.kernel(
        _gather_body,
        out_type=jax.ShapeDtypeStruct((ROWS, E), jnp.float32),
        mesh=plsc.VectorSubcoreMesh(core_axis_name="c", subcore_axis_name="s"),
        scratch_types=[
            pltpu.VMEM((CHUNKS, CHUNK), jnp.int32),
            pltpu.VMEM((NBUF, CHUNK, E), jnp.float32),
        ] + [pltpu.SemaphoreType.DMA] * NBUF,
        compiler_params=pltpu.CompilerParams(use_tc_tiling_on_sc=False),
    )


def _lstm_body(emb_ref, wih_ref, whh_ref, b_ref, out_ref, h_scr, c_scr):
    t = pl.program_id(1)

    @pl.when(t == 0)
    def _():
        h_scr[...] = jnp.zeros_like(h_scr)
        c_scr[...] = jnp.zeros_like(c_scr)

    xt = emb_ref[0]
    h = h_scr[...]
    gates = (
        lax.dot_general(xt, wih_ref[...], (((1,), (1,)), ((), ())),
                        preferred_element_type=jnp.float32)
        + lax.dot_general(h, whh_ref[...], (((1,), (1,)), ((), ())),
                          preferred_element_type=jnp.float32)
        + b_ref[...]
    )
    i = jax.nn.sigmoid(gates[:, 0:H])
    f = jax.nn.sigmoid(gates[:, H:2 * H])
    g = jnp.tanh(gates[:, 2 * H:3 * H])
    o = jax.nn.sigmoid(gates[:, 3 * H:4 * H])
    c_new = f * c_scr[...] + i * g
    h_new = o * jnp.tanh(c_new)
    c_scr[...] = c_new
    h_scr[...] = h_new

    @pl.when(t == T - 1)
    def _():
        out_ref[...] = h_new


def _lstm(emb_tm, W_ih, W_hh, bias, interpret=False):
    return pl.pallas_call(
        _lstm_body,
        grid=(NB, T),
        in_specs=[
            pl.BlockSpec((1, BT_TILE, E), lambda b, t: (t, b, 0)),
            pl.BlockSpec((4 * H, E), lambda b, t: (0, 0)),
            pl.BlockSpec((4 * H, H), lambda b, t: (0, 0)),
            pl.BlockSpec((1, 4 * H), lambda b, t: (0, 0)),
        ],
        out_specs=pl.BlockSpec((BT_TILE, H), lambda b, t: (b, 0)),
        out_shape=jax.ShapeDtypeStruct((B, H), jnp.float32),
        scratch_shapes=[
            pltpu.VMEM((BT_TILE, H), jnp.float32),
            pltpu.VMEM((BT_TILE, H), jnp.float32),
        ],
        compiler_params=pltpu.CompilerParams(
            dimension_semantics=("arbitrary", "arbitrary")),
        interpret=interpret,
    )(emb_tm, W_ih, W_hh, bias)


def kernel(x, emb, W_ih, W_hh, b_ih, b_hh):
    idx = jnp.transpose(x).astype(jnp.int32).reshape(NW, CHUNKS, CHUNK)
    emb_tm = _make_sc_gather()(emb, idx).reshape(T, B, E)
    bias = (b_ih + b_hh).reshape(1, 4 * H)
    return _lstm(emb_tm, W_ih, W_hh, bias)


# speedup vs baseline: 1.0342x; 1.0342x over previous
import functools

import jax
import jax.numpy as jnp
from jax import lax
from jax.experimental import pallas as pl
from jax.experimental.pallas import tpu as pltpu
from jax.experimental.pallas import tpu_sc as plsc

B, T = 4096, 200
V, E, H = 1000000, 64, 32

NC, NS = 2, 16
NW = NC * NS
BSTRIPE = B // NW
CHUNK = BSTRIPE
NBUF = 4
LANES = 16

BT_TILE = 4096
NB = B // BT_TILE


def _gather_body(table_hbm, x_hbm, out_hbm, idx_v, idxc_v, rows_v, *sems):
    wid = lax.axis_index("s") * NC + lax.axis_index("c")
    stripe = wid * BSTRIPE

    pltpu.sync_copy(x_hbm.at[pl.ds(stripe * T, BSTRIPE * T)], idx_v)

    def build_col(t, slot):
        for j in range(BSTRIPE // LANES):
            pos = (lax.iota(jnp.int32, LANES) + LANES * j) * T + t
            idxc_v[slot, pl.ds(LANES * j, LANES)] = plsc.load_gather(
                idx_v, [pos])

    def gather_copy(slot):
        return pltpu.make_async_copy(
            table_hbm.at[idxc_v.at[slot]], rows_v.at[slot], sems[slot])

    for b in range(NBUF):
        build_col(b, b)
        gather_copy(b).start()

    def body(i, carry):
        for b in range(NBUF):
            t = i * NBUF + b
            gather_copy(b).wait()
            pltpu.sync_copy(
                rows_v.at[b], out_hbm.at[pl.ds(t * B + stripe, CHUNK)])
            nxt = t + NBUF

            @pl.when(nxt < T)
            def _():
                build_col(nxt, b)
                gather_copy(b).start()
        return carry

    lax.fori_loop(0, T // NBUF, body, 0)


@functools.cache
def _make_sc_gather():
    return pl.kernel(
        _gather_body,
        out_type=jax.ShapeDtypeStruct((B * T, E), jnp.float32),
        mesh=plsc.VectorSubcoreMesh(core_axis_name="c", subcore_axis_name="s"),
        scratch_types=[
            pltpu.VMEM((BSTRIPE * T,), jnp.int32),
            pltpu.VMEM((NBUF, CHUNK), jnp.int32),
            pltpu.VMEM((NBUF, CHUNK, E), jnp.float32),
        ] + [pltpu.SemaphoreType.DMA] * NBUF,
        compiler_params=pltpu.CompilerParams(
            use_tc_tiling_on_sc=False, needs_layout_passes=False),
    )


def _lstm_body(emb_ref, wx_ref, wh_ref, b_ref, out_ref, h_scr, c_scr):
    t = pl.program_id(1)

    @pl.when(t == 0)
    def _():
        h_scr[...] = jnp.zeros_like(h_scr)
        c_scr[...] = jnp.zeros_like(c_scr)

    xt = emb_ref[0]
    h = h_scr[...]

    def gate(k):
        return (
            lax.dot_general(xt, wx_ref[k], (((1,), (0,)), ((), ())),
                            preferred_element_type=jnp.float32)
            + lax.dot_general(h, wh_ref[k], (((1,), (0,)), ((), ())),
                              preferred_element_type=jnp.float32)
            + b_ref[k]
        )

    i = jax.nn.sigmoid(gate(0))
    f = jax.nn.sigmoid(gate(1))
    g = jnp.tanh(gate(2))
    o = jax.nn.sigmoid(gate(3))
    c_new = f * c_scr[...] + i * g
    h_new = o * jnp.tanh(c_new)
    c_scr[...] = c_new
    h_scr[...] = h_new

    @pl.when(t == T - 1)
    def _():
        out_ref[...] = h_new


def _lstm(emb_tm, wx, wh, bias, interpret=False):
    return pl.pallas_call(
        _lstm_body,
        grid=(NB, T),
        in_specs=[
            pl.BlockSpec((1, BT_TILE, E), lambda b, t: (t, b, 0)),
            pl.BlockSpec((4, E, H), lambda b, t: (0, 0, 0)),
            pl.BlockSpec((4, H, H), lambda b, t: (0, 0, 0)),
            pl.BlockSpec((4, 1, H), lambda b, t: (0, 0, 0)),
        ],
        out_specs=pl.BlockSpec((BT_TILE, H), lambda b, t: (b, 0)),
        out_shape=jax.ShapeDtypeStruct((B, H), jnp.float32),
        scratch_shapes=[
            pltpu.VMEM((BT_TILE, H), jnp.float32),
            pltpu.VMEM((BT_TILE, H), jnp.float32),
        ],
        compiler_params=pltpu.CompilerParams(
            dimension_semantics=("arbitrary", "arbitrary")),
        interpret=interpret,
    )(emb_tm, wx, wh, bias)


def kernel(x, emb, W_ih, W_hh, b_ih, b_hh):
    emb_tm = _make_sc_gather()(
        emb, x.astype(jnp.int32).reshape(B * T)).reshape(T, B, E)
    wx = jnp.transpose(W_ih.reshape(4, H, E), (0, 2, 1))
    wh = jnp.transpose(W_hh.reshape(4, H, H), (0, 2, 1))
    bias = (b_ih + b_hh).reshape(4, 1, H)
    return _lstm(emb_tm, wx, wh, bias)


# speedup vs baseline: 1.6142x; 1.5608x over previous
import functools

import jax
import jax.numpy as jnp
from jax import lax
from jax.experimental import pallas as pl
from jax.experimental.pallas import tpu as pltpu
from jax.experimental.pallas import tpu_sc as plsc

B, T = 4096, 200
V, E, H = 1000000, 64, 32

NC, NS = 2, 16
NW = NC * NS
BSTRIPE = B // NW
CHUNK = BSTRIPE
NBUF = 4
LANES = 16

BT_TILE = 4096
NB = B // BT_TILE


def _gather_body(table_hbm, x_hbm, out_hbm, idx_v, idxc_v, rows_v, *sems):
    wid = lax.axis_index("s") * NC + lax.axis_index("c")
    stripe = wid * BSTRIPE

    pltpu.sync_copy(x_hbm.at[pl.ds(stripe * T, BSTRIPE * T)], idx_v)

    def build_col(t, slot):
        for j in range(BSTRIPE // LANES):
            pos = (lax.iota(jnp.int32, LANES) + LANES * j) * T + t
            idxc_v[slot, pl.ds(LANES * j, LANES)] = plsc.load_gather(
                idx_v, [pos])

    def gather_copy(slot):
        return pltpu.make_async_copy(
            table_hbm.at[idxc_v.at[slot]], rows_v.at[slot], sems[slot])

    for b in range(NBUF):
        build_col(b, b)
        gather_copy(b).start()

    def body(i, carry):
        for b in range(NBUF):
            t = i * NBUF + b
            gather_copy(b).wait()
            pltpu.sync_copy(
                rows_v.at[b], out_hbm.at[pl.ds(t * B + stripe, CHUNK)])
            nxt = t + NBUF

            @pl.when(nxt < T)
            def _():
                build_col(nxt, b)
                gather_copy(b).start()
        return carry

    lax.fori_loop(0, T // NBUF, body, 0)


@functools.cache
def _make_sc_gather():
    return pl.kernel(
        _gather_body,
        out_type=jax.ShapeDtypeStruct((B * T, E), jnp.float32),
        mesh=plsc.VectorSubcoreMesh(core_axis_name="c", subcore_axis_name="s"),
        scratch_types=[
            pltpu.VMEM((BSTRIPE * T,), jnp.int32),
            pltpu.VMEM((NBUF, CHUNK), jnp.int32),
            pltpu.VMEM((NBUF, CHUNK, E), jnp.float32),
        ] + [pltpu.SemaphoreType.DMA] * NBUF,
        compiler_params=pltpu.CompilerParams(
            use_tc_tiling_on_sc=False, needs_layout_passes=False),
    )


B2 = B // 2
E2, H2 = 2 * E, 2 * H


def _lstm_body(emb_ref, wx_ref, wh_ref, b_ref, out_ref, h_scr, c_scr):
    t = pl.program_id(0)

    @pl.when(t == 0)
    def _():
        h_scr[...] = jnp.zeros_like(h_scr)
        c_scr[...] = jnp.zeros_like(c_scr)

    xt = emb_ref[0]
    h = h_scr[...]

    def gate(k):
        return (
            lax.dot_general(xt, wx_ref[k], (((1,), (0,)), ((), ())),
                            preferred_element_type=jnp.float32)
            + lax.dot_general(h, wh_ref[k], (((1,), (0,)), ((), ())),
                              preferred_element_type=jnp.float32)
            + b_ref[k]
        )

    i = jax.nn.sigmoid(gate(0))
    f = jax.nn.sigmoid(gate(1))
    g = jnp.tanh(gate(2))
    o = jax.nn.sigmoid(gate(3))
    c_new = f * c_scr[...] + i * g
    h_new = o * jnp.tanh(c_new)
    c_scr[...] = c_new
    h_scr[...] = h_new

    @pl.when(t == T - 1)
    def _():
        out_ref[...] = h_new


def _lstm(emb_p, wx2, wh2, bias2, interpret=False):
    return pl.pallas_call(
        _lstm_body,
        grid=(T,),
        in_specs=[
            pl.BlockSpec((1, B2, E2), lambda t: (t, 0, 0)),
            pl.BlockSpec((4, E2, H2), lambda t: (0, 0, 0)),
            pl.BlockSpec((4, H2, H2), lambda t: (0, 0, 0)),
            pl.BlockSpec((4, 1, H2), lambda t: (0, 0, 0)),
        ],
        out_specs=pl.BlockSpec((B2, H2), lambda t: (0, 0)),
        out_shape=jax.ShapeDtypeStruct((B2, H2), jnp.float32),
        scratch_shapes=[
            pltpu.VMEM((B2, H2), jnp.float32),
            pltpu.VMEM((B2, H2), jnp.float32),
        ],
        compiler_params=pltpu.CompilerParams(
            dimension_semantics=("arbitrary",)),
        interpret=interpret,
    )(emb_p, wx2, wh2, bias2)


def _blockdiag(w):
    k4, K, Hh = w.shape
    z = jnp.zeros((k4, K, Hh), w.dtype)
    top = jnp.concatenate([w, z], axis=2)
    bot = jnp.concatenate([z, w], axis=2)
    return jnp.concatenate([top, bot], axis=1)


def kernel(x, emb, W_ih, W_hh, b_ih, b_hh):
    emb_p = _make_sc_gather()(
        emb, x.astype(jnp.int32).reshape(B * T)).reshape(T, B2, E2)
    wx = jnp.transpose(W_ih.reshape(4, H, E), (0, 2, 1))
    wh = jnp.transpose(W_hh.reshape(4, H, H), (0, 2, 1))
    bias = (b_ih + b_hh).reshape(4, 1, H)
    bias2 = jnp.concatenate([bias, bias], axis=2)
    out = _lstm(emb_p, _blockdiag(wx), _blockdiag(wh), bias2)
    return out.reshape(B, H)
